# baseline (speedup 1.0000x reference)
import functools

import jax
import jax.numpy as jnp
from jax import lax
from jax.experimental import pallas as pl
from jax.experimental.pallas import tpu as pltpu
from jax.experimental.pallas import tpu_sc as plsc

N = 10000
E = 320000
D = 128
OUT = 64

NP = 10240
NC = 2
NS = 16
NW = NC * NS
EPW = E // NW
K = 80
NCHUNK = EPW // K
RPT = NP // NS

_SC_MESH = plsc.VectorSubcoreMesh(core_axis_name="c", subcore_axis_name="s")

_SC_SCRATCH = [
    pltpu.VMEM((EPW,), jnp.int32),
    pltpu.VMEM((EPW,), jnp.int32),
    pltpu.VMEM((K,), jnp.int32),
    pltpu.VMEM((K,), jnp.int32),
    pltpu.VMEM((K,), jnp.int32),
    pltpu.VMEM((K,), jnp.int32),
    pltpu.VMEM((K, D), jnp.float32),
    pltpu.VMEM((K, D), jnp.float32),
    pltpu.VMEM((RPT,), jnp.float32),
    pltpu.VMEM((K,), jnp.float32),
    pltpu.VMEM_SHARED((NP, D), jnp.float32),
    pltpu.VMEM_SHARED((NP,), jnp.float32),
    pltpu.SemaphoreType.DMA,
    pltpu.SemaphoreType.DMA,
]


def _sc_agg_body(with_deg, x_hbm, src_hbm, dst_hbm, agg_out, deg_out,
                 src_all, dst_all, srcA, srcB, dstA, dstB, rowsA, rowsB,
                 stage1_v, ones_v, acc_sh, deg_sh, semA, semB):
    c = lax.axis_index("c")
    s = lax.axis_index("s")
    wid = c * NS + s
    zero16 = jnp.zeros((16,), jnp.float32)
    one16 = jnp.ones((16,), jnp.float32)

    def _zero_rows(i, carry):
        rowsA[i // 8, pl.ds((i % 8) * 16, 16)] = zero16
        return carry

    lax.fori_loop(0, K * (D // 16), _zero_rows, 0)

    if with_deg:
        def _zero_s1(i, carry):
            stage1_v[pl.ds(i * 16, 16)] = zero16
            return carry

        lax.fori_loop(0, RPT // 16, _zero_s1, 0)

        for i in range(K // 16):
            ones_v[pl.ds(i * 16, 16)] = one16

    tb = s * RPT
    ebase = pl.multiple_of(wid * EPW, 8)
    pltpu.async_copy(src_hbm.at[pl.ds(ebase, EPW)], src_all, semA)
    pltpu.async_copy(dst_hbm.at[pl.ds(ebase, EPW)], dst_all, semB)
    for j in range(RPT // K):
        pltpu.sync_copy(rowsA, acc_sh.at[pl.ds(tb + j * K, K)])
    if with_deg:
        pltpu.sync_copy(stage1_v, deg_sh.at[pl.ds(tb, RPT)])
    pltpu.make_async_copy(src_hbm.at[pl.ds(ebase, EPW)], src_all, semA).wait()
    pltpu.make_async_copy(dst_hbm.at[pl.ds(ebase, EPW)], dst_all, semB).wait()
    plsc.subcore_barrier()

    def _stage(lo, sbuf, dbuf):
        for i in range(K // 16):
            sbuf[pl.ds(i * 16, 16)] = src_all[pl.ds(lo + i * 16, 16)]
            dbuf[pl.ds(i * 16, 16)] = dst_all[pl.ds(lo + i * 16, 16)]

    def _scatter(rows, dbuf):
        pltpu.sync_copy(rows, acc_sh.at[dbuf], add=True)
        if with_deg:
            pltpu.sync_copy(ones_v, deg_sh.at[dbuf], add=True)

    _stage(0, srcA, dstA)
    pltpu.async_copy(x_hbm.at[srcA], rowsA, semA)

    def _pair(j, carry):
        lo0 = 2 * j * K
        pltpu.make_async_copy(x_hbm.at[srcA], rowsA, semA).wait()
        _stage(lo0 + K, srcB, dstB)
        pltpu.async_copy(x_hbm.at[srcB], rowsB, semB)
        _scatter(rowsA, dstA)
        pltpu.make_async_copy(x_hbm.at[srcB], rowsB, semB).wait()
        _stage(lo0 + 2 * K, srcA, dstA)
        pltpu.async_copy(x_hbm.at[srcA], rowsA, semA)
        _scatter(rowsB, dstB)
        return carry

    lax.fori_loop(0, (NCHUNK - 1) // 2, _pair, 0)
    pltpu.make_async_copy(x_hbm.at[srcA], rowsA, semA).wait()
    _scatter(rowsA, dstA)
    plsc.subcore_barrier()

    for j in range(RPT // K):
        r0 = tb + j * K
        pltpu.sync_copy(acc_sh.at[pl.ds(r0, K)], rowsA)
        pltpu.sync_copy(rowsA, agg_out.at[c, pl.ds(r0, K)])
    if with_deg:
        pltpu.sync_copy(deg_sh.at[pl.ds(tb, RPT)], stage1_v)
        pltpu.sync_copy(stage1_v, deg_out.at[c, pl.ds(tb, RPT)])


@functools.partial(
    pl.kernel,
    out_type=[
        jax.ShapeDtypeStruct((NC, NP, D), jnp.float32),
        jax.ShapeDtypeStruct((NC, NP), jnp.float32),
    ],
    mesh=_SC_MESH,
    scratch_types=_SC_SCRATCH,
)
def _sc_agg(x_hbm, src_hbm, dst_hbm, agg_out, deg_out, *scratch):
    _sc_agg_body(True, x_hbm, src_hbm, dst_hbm, agg_out, deg_out, *scratch)


@functools.partial(
    pl.kernel,
    out_type=jax.ShapeDtypeStruct((NC, NP, D), jnp.float32),
    mesh=_SC_MESH,
    scratch_types=_SC_SCRATCH,
)
def _sc_agg_nodeg(x_hbm, src_hbm, dst_hbm, agg_out, *scratch):
    _sc_agg_body(False, x_hbm, src_hbm, dst_hbm, agg_out, None, *scratch)


_BLK = 2000


def _layer1_body(x_ref, agg_ref, deg_ref, ws_ref, wn_ref, b_ref,
                 out_ref, scale_ref):
    deg = deg_ref[0] + deg_ref[1]
    scale = 1.0 / jnp.maximum(deg, 1.0)
    scale_ref[...] = scale
    aggs = agg_ref[0] + agg_ref[1]
    hn = aggs * scale
    h = (jnp.dot(x_ref[...], ws_ref[...], preferred_element_type=jnp.float32)
         + jnp.dot(hn, wn_ref[...], preferred_element_type=jnp.float32)
         + b_ref[...])
    out_ref[...] = jnp.maximum(h, 0.0)


_tc_layer1 = pl.pallas_call(
    _layer1_body,
    grid=(N // _BLK,),
    in_specs=[
        pl.BlockSpec((_BLK, D), lambda i: (i, 0)),
        pl.BlockSpec((NC, _BLK, D), lambda i: (0, i, 0)),
        pl.BlockSpec((NC, _BLK, 1), lambda i: (0, i, 0)),
        pl.BlockSpec((D, D), lambda i: (0, 0)),
        pl.BlockSpec((D, D), lambda i: (0, 0)),
        pl.BlockSpec((1, D), lambda i: (0, 0)),
    ],
    out_specs=[
        pl.BlockSpec((_BLK, D), lambda i: (i, 0)),
        pl.BlockSpec((_BLK, 1), lambda i: (i, 0)),
    ],
    out_shape=[
        jax.ShapeDtypeStruct((N, D), jnp.float32),
        jax.ShapeDtypeStruct((N, 1), jnp.float32),
    ],
)


def _layer2_body(h_ref, agg_ref, scale_ref, ws_ref, wn_ref, b_ref, wc_ref,
                 bc_ref, h2_ref, out_ref):
    aggs = agg_ref[0] + agg_ref[1]
    hn = aggs * scale_ref[...]
    h = (jnp.dot(h_ref[...], ws_ref[...], preferred_element_type=jnp.float32)
         + jnp.dot(hn, wn_ref[...], preferred_element_type=jnp.float32)
         + b_ref[...])
    h2 = jnp.maximum(h, 0.0)
    h2_ref[...] = h2
    out_ref[...] = (jnp.dot(h2, wc_ref[...], preferred_element_type=jnp.float32)
                    + bc_ref[...])


_tc_layer2 = pl.pallas_call(
    _layer2_body,
    grid=(N // _BLK,),
    in_specs=[
        pl.BlockSpec((_BLK, D), lambda i: (i, 0)),
        pl.BlockSpec((NC, _BLK, D), lambda i: (0, i, 0)),
        pl.BlockSpec((_BLK, 1), lambda i: (i, 0)),
        pl.BlockSpec((D, D), lambda i: (0, 0)),
        pl.BlockSpec((D, D), lambda i: (0, 0)),
        pl.BlockSpec((1, D), lambda i: (0, 0)),
        pl.BlockSpec((D, OUT), lambda i: (0, 0)),
        pl.BlockSpec((1, OUT), lambda i: (0, 0)),
    ],
    out_specs=[
        pl.BlockSpec((_BLK, D), lambda i: (i, 0)),
        pl.BlockSpec((_BLK, OUT), lambda i: (i, 0)),
    ],
    out_shape=[
        jax.ShapeDtypeStruct((N, D), jnp.float32),
        jax.ShapeDtypeStruct((N, OUT), jnp.float32),
    ],
)


def kernel(x, edge_index, W_self1, W_neigh1, b1, W_self2, W_neigh2, b2,
           W_cls, b_cls):
    src = edge_index[0]
    dst = edge_index[1]
    agg1, deg = _sc_agg(x, src, dst)
    h1, scale = _tc_layer1(x, agg1, deg.reshape(NC, NP, 1),
                           W_self1, W_neigh1, b1.reshape(1, D))
    agg2 = _sc_agg_nodeg(h1, src, dst)
    h2, logits = _tc_layer2(h1, agg2, scale, W_self2, W_neigh2,
                            b2.reshape(1, D), W_cls, b_cls.reshape(1, OUT))
    return (logits, h2)


# speedup vs baseline: 1.0003x; 1.0003x over previous
import functools

import jax
import jax.numpy as jnp
from jax import lax
from jax.experimental import pallas as pl
from jax.experimental.pallas import tpu as pltpu
from jax.experimental.pallas import tpu_sc as plsc

N = 10000
E = 320000
D = 128
OUT = 64

NP = 10240
NC = 2
NS = 16
NW = NC * NS
EPW = E // NW
K = 80
NCHUNK = EPW // K
RPT = NP // NS

_SC_MESH = plsc.VectorSubcoreMesh(core_axis_name="c", subcore_axis_name="s")

_SC_SCRATCH = [
    pltpu.VMEM((EPW,), jnp.int32),
    pltpu.VMEM((EPW,), jnp.int32),
    pltpu.VMEM((K,), jnp.int32),
    pltpu.VMEM((K,), jnp.int32),
    pltpu.VMEM((K,), jnp.int32),
    pltpu.VMEM((K,), jnp.int32),
    pltpu.VMEM((K, D), jnp.float32),
    pltpu.VMEM((K, D), jnp.float32),
    pltpu.VMEM((RPT,), jnp.float32),
    pltpu.VMEM((K,), jnp.float32),
    pltpu.VMEM_SHARED((NP, D), jnp.float32),
    pltpu.VMEM_SHARED((NP,), jnp.float32),
    pltpu.SemaphoreType.DMA,
    pltpu.SemaphoreType.DMA,
]


def _sc_agg_body(with_deg, x_hbm, src_hbm, dst_hbm, agg_out, deg_out,
                 src_all, dst_all, srcA, srcB, dstA, dstB, rowsA, rowsB,
                 stage1_v, ones_v, acc_sh, deg_sh, semA, semB):
    c = lax.axis_index("c")
    s = lax.axis_index("s")
    wid = c * NS + s
    zero16 = jnp.zeros((16,), jnp.float32)
    one16 = jnp.ones((16,), jnp.float32)

    def _zero_rows(i, carry):
        rowsA[i // 8, pl.ds((i % 8) * 16, 16)] = zero16
        return carry

    lax.fori_loop(0, K * (D // 16), _zero_rows, 0)

    if with_deg:
        def _zero_s1(i, carry):
            stage1_v[pl.ds(i * 16, 16)] = zero16
            return carry

        lax.fori_loop(0, RPT // 16, _zero_s1, 0)

        for i in range(K // 16):
            ones_v[pl.ds(i * 16, 16)] = one16

    tb = s * RPT
    ebase = pl.multiple_of(wid * EPW, 8)
    pltpu.async_copy(src_hbm.at[pl.ds(ebase, EPW)], src_all, semA)
    pltpu.async_copy(dst_hbm.at[pl.ds(ebase, EPW)], dst_all, semB)
    for j in range(RPT // K):
        pltpu.sync_copy(rowsA, acc_sh.at[pl.ds(tb + j * K, K)])
    if with_deg:
        pltpu.sync_copy(stage1_v, deg_sh.at[pl.ds(tb, RPT)])
    pltpu.make_async_copy(src_hbm.at[pl.ds(ebase, EPW)], src_all, semA).wait()
    pltpu.make_async_copy(dst_hbm.at[pl.ds(ebase, EPW)], dst_all, semB).wait()
    plsc.subcore_barrier()

    def _stage(lo, sbuf, dbuf):
        for i in range(K // 16):
            sbuf[pl.ds(i * 16, 16)] = src_all[pl.ds(lo + i * 16, 16)]
            dbuf[pl.ds(i * 16, 16)] = dst_all[pl.ds(lo + i * 16, 16)]

    def _scatter(rows, dbuf):
        pltpu.sync_copy(rows, acc_sh.at[dbuf], add=True)
        if with_deg:
            pltpu.sync_copy(ones_v, deg_sh.at[dbuf], add=True)

    _stage(0, srcA, dstA)
    pltpu.async_copy(x_hbm.at[srcA], rowsA, semA)

    def _pair(j, carry):
        lo0 = 2 * j * K
        pltpu.make_async_copy(x_hbm.at[srcA], rowsA, semA).wait()
        _stage(lo0 + K, srcB, dstB)
        pltpu.async_copy(x_hbm.at[srcB], rowsB, semB)
        _scatter(rowsA, dstA)
        pltpu.make_async_copy(x_hbm.at[srcB], rowsB, semB).wait()
        _stage(lo0 + 2 * K, srcA, dstA)
        pltpu.async_copy(x_hbm.at[srcA], rowsA, semA)
        _scatter(rowsB, dstB)
        return carry

    lax.fori_loop(0, (NCHUNK - 1) // 2, _pair, 0)
    pltpu.make_async_copy(x_hbm.at[srcA], rowsA, semA).wait()
    _scatter(rowsA, dstA)
    plsc.subcore_barrier()

    for j in range(RPT // K):
        r0 = tb + j * K
        pltpu.sync_copy(acc_sh.at[pl.ds(r0, K)], rowsA)
        pltpu.sync_copy(rowsA, agg_out.at[c, pl.ds(r0, K)])
    if with_deg:
        pltpu.sync_copy(deg_sh.at[pl.ds(tb, RPT)], stage1_v)
        pltpu.sync_copy(stage1_v, deg_out.at[c, pl.ds(tb, RPT)])


@functools.partial(
    pl.kernel,
    out_type=[
        jax.ShapeDtypeStruct((NC, NP, D), jnp.float32),
        jax.ShapeDtypeStruct((NC, NP), jnp.float32),
    ],
    mesh=_SC_MESH,
    scratch_types=_SC_SCRATCH,
)
def _sc_agg(x_hbm, src_hbm, dst_hbm, agg_out, deg_out, *scratch):
    _sc_agg_body(True, x_hbm, src_hbm, dst_hbm, agg_out, deg_out, *scratch)


@functools.partial(
    pl.kernel,
    out_type=jax.ShapeDtypeStruct((NC, NP, D), jnp.float32),
    mesh=_SC_MESH,
    scratch_types=_SC_SCRATCH,
)
def _sc_agg_nodeg(x_hbm, src_hbm, dst_hbm, agg_out, *scratch):
    _sc_agg_body(False, x_hbm, src_hbm, dst_hbm, agg_out, None, *scratch)


_BLK = 2000


def _self_body(x_ref, w_ref, b_ref, out_ref):
    out_ref[...] = (jnp.dot(x_ref[...], w_ref[...],
                            preferred_element_type=jnp.float32)
                    + b_ref[...])


_tc_self = pl.pallas_call(
    _self_body,
    grid=(N // _BLK,),
    in_specs=[
        pl.BlockSpec((_BLK, D), lambda i: (i, 0)),
        pl.BlockSpec((D, D), lambda i: (0, 0)),
        pl.BlockSpec((1, D), lambda i: (0, 0)),
    ],
    out_specs=pl.BlockSpec((_BLK, D), lambda i: (i, 0)),
    out_shape=jax.ShapeDtypeStruct((N, D), jnp.float32),
)


def _layer1_body(xs_ref, agg_ref, deg_ref, wn_ref, out_ref, scale_ref):
    deg = deg_ref[0] + deg_ref[1]
    scale = 1.0 / jnp.maximum(deg, 1.0)
    scale_ref[...] = scale
    aggs = agg_ref[0] + agg_ref[1]
    hn = aggs * scale
    h = xs_ref[...] + jnp.dot(hn, wn_ref[...],
                              preferred_element_type=jnp.float32)
    out_ref[...] = jnp.maximum(h, 0.0)


_tc_layer1 = pl.pallas_call(
    _layer1_body,
    grid=(N // _BLK,),
    in_specs=[
        pl.BlockSpec((_BLK, D), lambda i: (i, 0)),
        pl.BlockSpec((NC, _BLK, D), lambda i: (0, i, 0)),
        pl.BlockSpec((NC, _BLK, 1), lambda i: (0, i, 0)),
        pl.BlockSpec((D, D), lambda i: (0, 0)),
    ],
    out_specs=[
        pl.BlockSpec((_BLK, D), lambda i: (i, 0)),
        pl.BlockSpec((_BLK, 1), lambda i: (i, 0)),
    ],
    out_shape=[
        jax.ShapeDtypeStruct((N, D), jnp.float32),
        jax.ShapeDtypeStruct((N, 1), jnp.float32),
    ],
)


def _layer2_body(hs_ref, agg_ref, scale_ref, wn_ref, wc_ref,
                 bc_ref, h2_ref, out_ref):
    aggs = agg_ref[0] + agg_ref[1]
    hn = aggs * scale_ref[...]
    h = hs_ref[...] + jnp.dot(hn, wn_ref[...],
                              preferred_element_type=jnp.float32)
    h2 = jnp.maximum(h, 0.0)
    h2_ref[...] = h2
    out_ref[...] = (jnp.dot(h2, wc_ref[...], preferred_element_type=jnp.float32)
                    + bc_ref[...])


_tc_layer2 = pl.pallas_call(
    _layer2_body,
    grid=(N // _BLK,),
    in_specs=[
        pl.BlockSpec((_BLK, D), lambda i: (i, 0)),
        pl.BlockSpec((NC, _BLK, D), lambda i: (0, i, 0)),
        pl.BlockSpec((_BLK, 1), lambda i: (i, 0)),
        pl.BlockSpec((D, D), lambda i: (0, 0)),
        pl.BlockSpec((D, OUT), lambda i: (0, 0)),
        pl.BlockSpec((1, OUT), lambda i: (0, 0)),
    ],
    out_specs=[
        pl.BlockSpec((_BLK, D), lambda i: (i, 0)),
        pl.BlockSpec((_BLK, OUT), lambda i: (i, 0)),
    ],
    out_shape=[
        jax.ShapeDtypeStruct((N, D), jnp.float32),
        jax.ShapeDtypeStruct((N, OUT), jnp.float32),
    ],
)


def kernel(x, edge_index, W_self1, W_neigh1, b1, W_self2, W_neigh2, b2,
           W_cls, b_cls):
    src = edge_index[0]
    dst = edge_index[1]
    agg1, deg = _sc_agg(x, src, dst)
    xs1 = _tc_self(x, W_self1, b1.reshape(1, D))
    h1, scale = _tc_layer1(xs1, agg1, deg.reshape(NC, NP, 1), W_neigh1)
    agg2 = _sc_agg_nodeg(h1, src, dst)
    hs2 = _tc_self(h1, W_self2, b2.reshape(1, D))
    h2, logits = _tc_layer2(hs2, agg2, scale, W_neigh2,
                            W_cls, b_cls.reshape(1, OUT))
    return (logits, h2)


# speedup vs baseline: 1.0227x; 1.0223x over previous
import functools

import jax
import jax.numpy as jnp
from jax import lax
from jax.experimental import pallas as pl
from jax.experimental.pallas import tpu as pltpu
from jax.experimental.pallas import tpu_sc as plsc

N = 10000
E = 320000
D = 128
OUT = 64

NP = 10240
NC = 2
NS = 16
NW = NC * NS
EPW = E // NW
K = 80
NCHUNK = EPW // K
RPT = NP // NS

_SC_MESH = plsc.VectorSubcoreMesh(core_axis_name="c", subcore_axis_name="s")

_SC_SCRATCH = [
    pltpu.VMEM((EPW,), jnp.int32),
    pltpu.VMEM((EPW,), jnp.int32),
    pltpu.VMEM((K,), jnp.int32),
    pltpu.VMEM((K,), jnp.int32),
    pltpu.VMEM((K,), jnp.int32),
    pltpu.VMEM((K,), jnp.int32),
    pltpu.VMEM((K, D), jnp.float32),
    pltpu.VMEM((K, D), jnp.float32),
    pltpu.VMEM((RPT,), jnp.float32),
    pltpu.VMEM((K,), jnp.float32),
    pltpu.VMEM_SHARED((NP, D), jnp.float32),
    pltpu.VMEM_SHARED((NP,), jnp.float32),
    pltpu.SemaphoreType.DMA,
    pltpu.SemaphoreType.DMA,
]


def _sc_agg_body(with_deg, x_hbm, src_hbm, dst_hbm, agg_out, deg_out,
                 src_all, dst_all, srcA, srcB, dstA, dstB, rowsA, rowsB,
                 stage1_v, ones_v, acc_sh, deg_sh, semA, semB):
    c = lax.axis_index("c")
    s = lax.axis_index("s")
    wid = c * NS + s
    zero16 = jnp.zeros((16,), jnp.float32)
    one16 = jnp.ones((16,), jnp.float32)

    tb = s * RPT
    ebase = pl.multiple_of(wid * EPW, 8)
    pltpu.async_copy(src_hbm.at[pl.ds(ebase, EPW)], src_all, semA)
    pltpu.async_copy(dst_hbm.at[pl.ds(ebase, EPW)], dst_all, semB)

    def _zero_rows(i, carry):
        rowsA[i // 8, pl.ds((i % 8) * 16, 16)] = zero16
        return carry

    lax.fori_loop(0, K * (D // 16), _zero_rows, 0)

    if with_deg:
        def _zero_s1(i, carry):
            stage1_v[pl.ds(i * 16, 16)] = zero16
            return carry

        lax.fori_loop(0, RPT // 16, _zero_s1, 0)

        for i in range(K // 16):
            ones_v[pl.ds(i * 16, 16)] = one16

    for j in range(RPT // K):
        pltpu.sync_copy(rowsA, acc_sh.at[pl.ds(tb + j * K, K)])
    if with_deg:
        pltpu.sync_copy(stage1_v, deg_sh.at[pl.ds(tb, RPT)])
    pltpu.make_async_copy(src_hbm.at[pl.ds(ebase, EPW)], src_all, semA).wait()
    pltpu.make_async_copy(dst_hbm.at[pl.ds(ebase, EPW)], dst_all, semB).wait()
    plsc.subcore_barrier()

    def _scatter(rows, lo):
        pltpu.sync_copy(rows, acc_sh.at[dst_all.at[pl.ds(lo, K)]], add=True)
        if with_deg:
            pltpu.sync_copy(ones_v, deg_sh.at[dst_all.at[pl.ds(lo, K)]],
                            add=True)

    def _gather(lo, rows, sem):
        pltpu.async_copy(x_hbm.at[src_all.at[pl.ds(lo, K)]], rows, sem)

    def _gwait(lo, rows, sem):
        pltpu.make_async_copy(x_hbm.at[src_all.at[pl.ds(lo, K)]], rows,
                              sem).wait()

    _gather(0, rowsA, semA)

    def _pair(j, carry):
        lo0 = 2 * j * K
        _gwait(lo0, rowsA, semA)
        _gather(lo0 + K, rowsB, semB)
        _scatter(rowsA, lo0)
        _gwait(lo0 + K, rowsB, semB)
        _gather(lo0 + 2 * K, rowsA, semA)
        _scatter(rowsB, lo0 + K)
        return carry

    lax.fori_loop(0, (NCHUNK - 1) // 2, _pair, 0)
    _gwait((NCHUNK - 1) * K, rowsA, semA)
    _scatter(rowsA, (NCHUNK - 1) * K)
    plsc.subcore_barrier()

    for j in range(RPT // K):
        r0 = tb + j * K
        pltpu.sync_copy(acc_sh.at[pl.ds(r0, K)], rowsA)
        pltpu.sync_copy(rowsA, agg_out.at[c, pl.ds(r0, K)])
    if with_deg:
        pltpu.sync_copy(deg_sh.at[pl.ds(tb, RPT)], stage1_v)
        pltpu.sync_copy(stage1_v, deg_out.at[c, pl.ds(tb, RPT)])


@functools.partial(
    pl.kernel,
    out_type=[
        jax.ShapeDtypeStruct((NC, NP, D), jnp.float32),
        jax.ShapeDtypeStruct((NC, NP), jnp.float32),
    ],
    mesh=_SC_MESH,
    scratch_types=_SC_SCRATCH,
)
def _sc_agg(x_hbm, src_hbm, dst_hbm, agg_out, deg_out, *scratch):
    _sc_agg_body(True, x_hbm, src_hbm, dst_hbm, agg_out, deg_out, *scratch)


@functools.partial(
    pl.kernel,
    out_type=jax.ShapeDtypeStruct((NC, NP, D), jnp.float32),
    mesh=_SC_MESH,
    scratch_types=_SC_SCRATCH,
)
def _sc_agg_nodeg(x_hbm, src_hbm, dst_hbm, agg_out, *scratch):
    _sc_agg_body(False, x_hbm, src_hbm, dst_hbm, agg_out, None, *scratch)


_BLK = 2000


def _self_body(x_ref, w_ref, b_ref, out_ref):
    out_ref[...] = (jnp.dot(x_ref[...], w_ref[...],
                            preferred_element_type=jnp.float32)
                    + b_ref[...])


_tc_self = pl.pallas_call(
    _self_body,
    grid=(N // _BLK,),
    in_specs=[
        pl.BlockSpec((_BLK, D), lambda i: (i, 0)),
        pl.BlockSpec((D, D), lambda i: (0, 0)),
        pl.BlockSpec((1, D), lambda i: (0, 0)),
    ],
    out_specs=pl.BlockSpec((_BLK, D), lambda i: (i, 0)),
    out_shape=jax.ShapeDtypeStruct((N, D), jnp.float32),
)


def _layer1_body(xs_ref, agg_ref, deg_ref, wn_ref, out_ref, scale_ref):
    deg = deg_ref[0] + deg_ref[1]
    scale = 1.0 / jnp.maximum(deg, 1.0)
    scale_ref[...] = scale
    aggs = agg_ref[0] + agg_ref[1]
    hn = aggs * scale
    h = xs_ref[...] + jnp.dot(hn, wn_ref[...],
                              preferred_element_type=jnp.float32)
    out_ref[...] = jnp.maximum(h, 0.0)


_tc_layer1 = pl.pallas_call(
    _layer1_body,
    grid=(N // _BLK,),
    in_specs=[
        pl.BlockSpec((_BLK, D), lambda i: (i, 0)),
        pl.BlockSpec((NC, _BLK, D), lambda i: (0, i, 0)),
        pl.BlockSpec((NC, _BLK, 1), lambda i: (0, i, 0)),
        pl.BlockSpec((D, D), lambda i: (0, 0)),
    ],
    out_specs=[
        pl.BlockSpec((_BLK, D), lambda i: (i, 0)),
        pl.BlockSpec((_BLK, 1), lambda i: (i, 0)),
    ],
    out_shape=[
        jax.ShapeDtypeStruct((N, D), jnp.float32),
        jax.ShapeDtypeStruct((N, 1), jnp.float32),
    ],
)


def _layer2_body(hs_ref, agg_ref, scale_ref, wn_ref, wc_ref,
                 bc_ref, h2_ref, out_ref):
    aggs = agg_ref[0] + agg_ref[1]
    hn = aggs * scale_ref[...]
    h = hs_ref[...] + jnp.dot(hn, wn_ref[...],
                              preferred_element_type=jnp.float32)
    h2 = jnp.maximum(h, 0.0)
    h2_ref[...] = h2
    out_ref[...] = (jnp.dot(h2, wc_ref[...], preferred_element_type=jnp.float32)
                    + bc_ref[...])


_tc_layer2 = pl.pallas_call(
    _layer2_body,
    grid=(N // _BLK,),
    in_specs=[
        pl.BlockSpec((_BLK, D), lambda i: (i, 0)),
        pl.BlockSpec((NC, _BLK, D), lambda i: (0, i, 0)),
        pl.BlockSpec((_BLK, 1), lambda i: (i, 0)),
        pl.BlockSpec((D, D), lambda i: (0, 0)),
        pl.BlockSpec((D, OUT), lambda i: (0, 0)),
        pl.BlockSpec((1, OUT), lambda i: (0, 0)),
    ],
    out_specs=[
        pl.BlockSpec((_BLK, D), lambda i: (i, 0)),
        pl.BlockSpec((_BLK, OUT), lambda i: (i, 0)),
    ],
    out_shape=[
        jax.ShapeDtypeStruct((N, D), jnp.float32),
        jax.ShapeDtypeStruct((N, OUT), jnp.float32),
    ],
)


def kernel(x, edge_index, W_self1, W_neigh1, b1, W_self2, W_neigh2, b2,
           W_cls, b_cls):
    src = edge_index[0]
    dst = edge_index[1]
    agg1, deg = _sc_agg(x, src, dst)
    xs1 = _tc_self(x, W_self1, b1.reshape(1, D))
    h1, scale = _tc_layer1(xs1, agg1, deg.reshape(NC, NP, 1), W_neigh1)
    agg2 = _sc_agg_nodeg(h1, src, dst)
    hs2 = _tc_self(h1, W_self2, b2.reshape(1, D))
    h2, logits = _tc_layer2(hs2, agg2, scale, W_neigh2,
                            W_cls, b_cls.reshape(1, OUT))
    return (logits, h2)
